# baseline (speedup 1.0000x reference)
import functools

import jax
import jax.numpy as jnp
from jax import lax
from jax.experimental import pallas as pl
from jax.experimental.pallas import tpu as pltpu
from jax.experimental.pallas import tpu_sc as plsc

N = 10000
E = 160000
IN_DIM = 128
H = 256
HH = H // 2
EPS = 1e-5

NCORE = 2
NSUB = 16
RPT = 624
REM_BASE = RPT * NSUB
REM_ROWS = N - REM_BASE
ZROWS = 16
ZCOPIES = RPT // ZROWS

E_PAD = 163840
N_ACC = N + 16

AGG_CH = 40
AGG_EPT = E // NSUB
AGG_CHUNKS = AGG_EPT // AGG_CH
NSLOT = 5
AGG_STEPS = AGG_CHUNKS // NSLOT - 1

DEG_CH = 40
DEG_EPT = E // (NSUB * NCORE)
DEG_CHUNKS = DEG_EPT // DEG_CH

MXU_PREC = lax.Precision.HIGHEST


def _fill(buf, rows, width, vec):
    for j in range(rows):
        for k in range(width // 16):
            buf[j, pl.ds(k * 16, 16)] = vec


def _zero_acc(zbuf, acc, s):
    for k in range(ZCOPIES):
        pltpu.sync_copy(zbuf, acc.at[pl.ds(s * RPT + k * ZROWS, ZROWS)])

    @pl.when(s == NSUB - 1)
    def _():
        pltpu.sync_copy(zbuf.at[pl.ds(0, REM_ROWS)],
                        acc.at[pl.ds(REM_BASE, REM_ROWS)])


def _deg_body(dst_hbm, out_hbm, didx, ones, zbuf, dacc):
    c = lax.axis_index("c")
    s = lax.axis_index("s")
    _fill(ones, DEG_CH, HH, jnp.ones((16,), jnp.float32))
    _fill(zbuf, ZROWS, HH, jnp.zeros((16,), jnp.float32))
    _zero_acc(zbuf, dacc, s)
    plsc.subcore_barrier()
    ebase = (c * NSUB + s) * DEG_EPT

    def step(i, carry):
        b = ebase + i * DEG_CH
        pltpu.sync_copy(dst_hbm.at[pl.ds(b, DEG_CH)], didx)
        pltpu.sync_copy(ones, dacc.at[didx], add=True)
        return carry

    lax.fori_loop(0, DEG_CHUNKS, step, 0)
    plsc.subcore_barrier()
    pltpu.sync_copy(dacc.at[pl.ds(s * RPT, RPT)],
                    out_hbm.at[pl.ds(c * N + s * RPT, RPT)])

    @pl.when(s == NSUB - 1)
    def _():
        pltpu.sync_copy(dacc.at[pl.ds(REM_BASE, REM_ROWS)],
                        out_hbm.at[pl.ds(c * N + REM_BASE, REM_ROWS)])


def _agg_body(y0_hbm, y1_hbm, src_hbm, dst_hbm, out_hbm,
              si0, si1, si2, si3, si4, di0, di1, di2, di3, di4,
              r0, r1, r2, r3, r4, zbuf, acc,
              sg0, sg1, sg2, sg3, sg4):
    c = lax.axis_index("c")
    s = lax.axis_index("s")
    sis = [si0, si1, si2, si3, si4]
    dis = [di0, di1, di2, di3, di4]
    rows = [r0, r1, r2, r3, r4]
    sgs = [sg0, sg1, sg2, sg3, sg4]
    ebase = s * AGG_EPT

    def _load(g, b):
        base = ebase + g * AGG_CH
        pltpu.sync_copy(src_hbm.at[pl.ds(base, AGG_CH)], sis[b])
        pltpu.sync_copy(dst_hbm.at[pl.ds(base, AGG_CH)], dis[b])

    def _g_start(b):
        pass

    def _g_wait(b):
        @pl.when(c == 0)
        def _():
            pltpu.sync_copy(y0_hbm.at[sis[b]], rows[b])

        @pl.when(c == 1)
        def _():
            pltpu.sync_copy(y1_hbm.at[sis[b]], rows[b])

    def _scatter(b):
        pltpu.sync_copy(rows[b], acc.at[dis[b]], add=True)

    for b in range(NSLOT):
        _load(b, b)
        _g_start(b)
    _fill(zbuf, ZROWS, HH, jnp.zeros((16,), jnp.float32))
    _zero_acc(zbuf, acc, s)
    plsc.subcore_barrier()

    def step(t, carry):
        for b in range(NSLOT):
            _g_wait(b)
            _scatter(b)
            _load(t * NSLOT + b + NSLOT, b)
            _g_start(b)
        return carry

    lax.fori_loop(0, AGG_STEPS, step, 0)
    for b in range(NSLOT):
        _g_wait(b)
        _scatter(b)
    plsc.subcore_barrier()
    pltpu.sync_copy(acc.at[pl.ds(s * RPT, RPT)],
                    out_hbm.at[pl.ds(c * N + s * RPT, RPT)])

    @pl.when(s == NSUB - 1)
    def _():
        pltpu.sync_copy(acc.at[pl.ds(REM_BASE, REM_ROWS)],
                        out_hbm.at[pl.ds(c * N + REM_BASE, REM_ROWS)])


@functools.cache
def _sc_kernels():
    mesh = plsc.VectorSubcoreMesh(
        core_axis_name="c", subcore_axis_name="s",
        num_cores=NCORE, num_subcores=NSUB)
    deg = pl.kernel(
        _deg_body,
        out_type=jax.ShapeDtypeStruct((NCORE * N, HH), jnp.float32),
        mesh=mesh,
        scratch_types=[
            pltpu.VMEM((DEG_CH,), jnp.int32),
            pltpu.VMEM((DEG_CH, HH), jnp.float32),
            pltpu.VMEM((ZROWS, HH), jnp.float32),
            pltpu.VMEM_SHARED((N, HH), jnp.float32),
        ],
    )
    agg = pl.kernel(
        _agg_body,
        out_type=jax.ShapeDtypeStruct((NCORE * N, HH), jnp.float32),
        mesh=mesh,
        scratch_types=(
            [pltpu.VMEM((AGG_CH,), jnp.int32)] * (2 * NSLOT)
            + [pltpu.VMEM((AGG_CH, HH), jnp.float32)] * NSLOT
            + [pltpu.VMEM((ZROWS, HH), jnp.float32),
               pltpu.VMEM_SHARED((N, HH), jnp.float32)]
            + [pltpu.SemaphoreType.DMA] * NSLOT
        ),
    )
    return deg, agg


BLK = 1000
GRID = N // BLK


def _dinv(p_ref, q_ref):
    d = 1.0 + p_ref[:, 0:1] + q_ref[:, 0:1]
    return lax.rsqrt(d)


def _pre_body(x_ref, win_ref, bin_ref, w0_ref, p_ref, q_ref,
              h_ref, y0_ref, y1_ref):
    h = jnp.dot(x_ref[...], win_ref[...], preferred_element_type=jnp.float32,
                precision=MXU_PREC) + bin_ref[...]
    h_ref[...] = h
    dinv = _dinv(p_ref, q_ref)
    y = dinv * jnp.dot(h, w0_ref[...], preferred_element_type=jnp.float32,
                       precision=MXU_PREC)
    y0_ref[...] = y[:, :HH]
    y1_ref[...] = y[:, HH:]


def _update(h_ref, y0_ref, y1_ref, a0_ref, a1_ref, dinv, b_ref, g_ref, be_ref):
    aggy = jnp.concatenate([a0_ref[...] + y0_ref[...],
                            a1_ref[...] + y1_ref[...]], axis=1)
    u = h_ref[...] + dinv * aggy + b_ref[...]
    m = jnp.mean(u, axis=1, keepdims=True)
    v = jnp.mean((u - m) ** 2, axis=1, keepdims=True)
    hn = (u - m) * lax.rsqrt(v + EPS) * g_ref[...] + be_ref[...]
    return jnp.maximum(hn, 0.0)


def _mid_body(h_ref, y0_ref, y1_ref, a0_ref, a1_ref, p_ref, q_ref,
              b_ref, g_ref, be_ref, wn_ref, ho_ref, yo0_ref, yo1_ref):
    dinv = _dinv(p_ref, q_ref)
    h = _update(h_ref, y0_ref, y1_ref, a0_ref, a1_ref, dinv, b_ref, g_ref, be_ref)
    ho_ref[...] = h
    y = dinv * jnp.dot(h, wn_ref[...], preferred_element_type=jnp.float32,
                       precision=MXU_PREC)
    yo0_ref[...] = y[:, :HH]
    yo1_ref[...] = y[:, HH:]


def _post_body(h_ref, y0_ref, y1_ref, a0_ref, a1_ref, p_ref, q_ref,
               b_ref, g_ref, be_ref, wh1_ref, bh1_ref, wh2_ref, bh2_ref, o_ref):
    dinv = _dinv(p_ref, q_ref)
    h = _update(h_ref, y0_ref, y1_ref, a0_ref, a1_ref, dinv, b_ref, g_ref, be_ref)
    t = jnp.maximum(jnp.dot(h, wh1_ref[...], preferred_element_type=jnp.float32,
                            precision=MXU_PREC) + bh1_ref[...], 0.0)
    o = jnp.dot(t, wh2_ref[...], preferred_element_type=jnp.float32,
                precision=MXU_PREC) + bh2_ref[...]
    o_ref[...] = 1.0 / (1.0 + jnp.exp(-o))


def _row_spec(w):
    return pl.BlockSpec((BLK, w), lambda i: (i, 0))


def _hi_spec():
    return pl.BlockSpec((BLK, HH), lambda i: (i + GRID, 0))


def _full_spec(shape):
    nd = len(shape)
    return pl.BlockSpec(shape, lambda i, _nd=nd: (0,) * _nd)


_pre_call = pl.pallas_call(
    _pre_body,
    grid=(GRID,),
    in_specs=[_row_spec(IN_DIM), _full_spec((IN_DIM, H)), _full_spec((1, H)),
              _full_spec((H, H)), _row_spec(HH), _hi_spec()],
    out_specs=[_row_spec(H), _row_spec(HH), _row_spec(HH)],
    out_shape=[jax.ShapeDtypeStruct((N, H), jnp.float32),
               jax.ShapeDtypeStruct((N, HH), jnp.float32),
               jax.ShapeDtypeStruct((N, HH), jnp.float32)],
)

_mid_call = pl.pallas_call(
    _mid_body,
    grid=(GRID,),
    in_specs=[_row_spec(H), _row_spec(HH), _row_spec(HH), _row_spec(HH),
              _hi_spec(), _row_spec(HH), _hi_spec(),
              _full_spec((1, H)), _full_spec((1, H)), _full_spec((1, H)),
              _full_spec((H, H))],
    out_specs=[_row_spec(H), _row_spec(HH), _row_spec(HH)],
    out_shape=[jax.ShapeDtypeStruct((N, H), jnp.float32),
               jax.ShapeDtypeStruct((N, HH), jnp.float32),
               jax.ShapeDtypeStruct((N, HH), jnp.float32)],
)

_post_call = pl.pallas_call(
    _post_body,
    grid=(GRID,),
    in_specs=[_row_spec(H), _row_spec(HH), _row_spec(HH), _row_spec(HH),
              _hi_spec(), _row_spec(HH), _hi_spec(),
              _full_spec((1, H)), _full_spec((1, H)), _full_spec((1, H)),
              _full_spec((H, HH)), _full_spec((1, HH)),
              _full_spec((HH, 1)), _full_spec((1, 1))],
    out_specs=[_row_spec(1)],
    out_shape=[jax.ShapeDtypeStruct((N, 1), jnp.float32)],
)


def kernel(x, edge_index, W_in, b_in, W0, b0, g0, beta0, W1, b1, g1, beta1,
           W2, b2, g2, beta2, Wh1, bh1, Wh2, bh2):
    src = edge_index[0]
    dst = edge_index[1]
    _deg_kernel, _agg_kernel = _sc_kernels()

    p = _deg_kernel(dst)

    r1 = lambda a: a.reshape(1, -1)
    h0, y00, y01 = _pre_call(x, W_in, r1(b_in), W0, p, p)
    a0 = _agg_kernel(y00, y01, src, dst)
    h1, y10, y11 = _mid_call(h0, y00, y01, a0, a0, p, p,
                             r1(b0), r1(g0), r1(beta0), W1)
    a1 = _agg_kernel(y10, y11, src, dst)
    h2, y20, y21 = _mid_call(h1, y10, y11, a1, a1, p, p,
                             r1(b1), r1(g1), r1(beta1), W2)
    a2 = _agg_kernel(y20, y21, src, dst)
    (out,) = _post_call(h2, y20, y21, a2, a2, p, p,
                        r1(b2), r1(g2), r1(beta2),
                        Wh1, r1(bh1), Wh2, bh2.reshape(1, 1))
    return out[:, 0]


# speedup vs baseline: 1.5417x; 1.5417x over previous
import functools

import jax
import jax.numpy as jnp
from jax import lax
from jax.experimental import pallas as pl
from jax.experimental.pallas import tpu as pltpu
from jax.experimental.pallas import tpu_sc as plsc

N = 10000
E = 160000
IN_DIM = 128
H = 256
HH = H // 2
EPS = 1e-5

NCORE = 2
NSUB = 16
RPT = 624
REM_BASE = RPT * NSUB
REM_ROWS = N - REM_BASE
ZROWS = 16
ZCOPIES = RPT // ZROWS

E_PAD = 163840
N_ACC = N + 16

AGG_CH = 40
AGG_EPT = E // NSUB
AGG_CHUNKS = AGG_EPT // AGG_CH
NSLOT = 5
AGG_STEPS = AGG_CHUNKS // NSLOT - 1

DEG_CH = 40
DEG_EPT = E // (NSUB * NCORE)
DEG_CHUNKS = DEG_EPT // DEG_CH

MXU_PREC = lax.Precision.HIGHEST


def _fill(buf, rows, width, vec):
    for j in range(rows):
        for k in range(width // 16):
            buf[j, pl.ds(k * 16, 16)] = vec


def _zero_acc(zbuf, acc, s):
    for k in range(ZCOPIES):
        pltpu.sync_copy(zbuf, acc.at[pl.ds(s * RPT + k * ZROWS, ZROWS)])

    @pl.when(s == NSUB - 1)
    def _():
        pltpu.sync_copy(zbuf.at[pl.ds(0, REM_ROWS)],
                        acc.at[pl.ds(REM_BASE, REM_ROWS)])


def _deg_body(dst_hbm, out_hbm, didx, ones, zbuf, dacc):
    c = lax.axis_index("c")
    s = lax.axis_index("s")
    _fill(ones, DEG_CH, HH, jnp.ones((16,), jnp.float32))
    _fill(zbuf, ZROWS, HH, jnp.zeros((16,), jnp.float32))
    _zero_acc(zbuf, dacc, s)
    plsc.subcore_barrier()
    ebase = (c * NSUB + s) * DEG_EPT

    def step(i, carry):
        b = ebase + i * DEG_CH
        pltpu.sync_copy(dst_hbm.at[pl.ds(b, DEG_CH)], didx)
        pltpu.sync_copy(ones, dacc.at[didx], add=True)
        return carry

    lax.fori_loop(0, DEG_CHUNKS, step, 0)
    plsc.subcore_barrier()
    pltpu.sync_copy(dacc.at[pl.ds(s * RPT, RPT)],
                    out_hbm.at[pl.ds(c * N + s * RPT, RPT)])

    @pl.when(s == NSUB - 1)
    def _():
        pltpu.sync_copy(dacc.at[pl.ds(REM_BASE, REM_ROWS)],
                        out_hbm.at[pl.ds(c * N + REM_BASE, REM_ROWS)])


def _agg_body(y0_hbm, y1_hbm, src_hbm, dst_hbm, out_hbm,
              si0, si1, si2, si3, si4, di0, di1, di2, di3, di4,
              r0, r1, r2, r3, r4, zbuf, acc,
              sg0, sg1, sg2, sg3, sg4):
    c = lax.axis_index("c")
    s = lax.axis_index("s")
    sis = [si0, si1, si2, si3, si4]
    dis = [di0, di1, di2, di3, di4]
    rows = [r0, r1, r2, r3, r4]
    sgs = [sg0, sg1, sg2, sg3, sg4]
    ebase = s * AGG_EPT

    def _load(g, b):
        base = ebase + g * AGG_CH
        pltpu.sync_copy(src_hbm.at[pl.ds(base, AGG_CH)], sis[b])
        pltpu.sync_copy(dst_hbm.at[pl.ds(base, AGG_CH)], dis[b])

    def _g_start(b):
        @pl.when(c == 0)
        def _():
            pltpu.async_copy(y0_hbm.at[sis[b]], rows[b], sgs[b])

        @pl.when(c == 1)
        def _():
            pltpu.async_copy(y1_hbm.at[sis[b]], rows[b], sgs[b])

    def _g_wait(b):
        @pl.when(c == 0)
        def _():
            pltpu.make_async_copy(y0_hbm.at[sis[b]], rows[b], sgs[b]).wait()

        @pl.when(c == 1)
        def _():
            pltpu.make_async_copy(y1_hbm.at[sis[b]], rows[b], sgs[b]).wait()

    def _scatter(b):
        pltpu.sync_copy(rows[b], acc.at[dis[b]], add=True)

    for b in range(NSLOT):
        _load(b, b)
        _g_start(b)
    _fill(zbuf, ZROWS, HH, jnp.zeros((16,), jnp.float32))
    _zero_acc(zbuf, acc, s)
    plsc.subcore_barrier()

    def step(t, carry):
        for b in range(NSLOT):
            _g_wait(b)
            _scatter(b)
            _load(t * NSLOT + b + NSLOT, b)
            _g_start(b)
        return carry

    lax.fori_loop(0, AGG_STEPS, step, 0)
    for b in range(NSLOT):
        _g_wait(b)
        _scatter(b)
    plsc.subcore_barrier()
    pltpu.sync_copy(acc.at[pl.ds(s * RPT, RPT)],
                    out_hbm.at[pl.ds(c * N + s * RPT, RPT)])

    @pl.when(s == NSUB - 1)
    def _():
        pltpu.sync_copy(acc.at[pl.ds(REM_BASE, REM_ROWS)],
                        out_hbm.at[pl.ds(c * N + REM_BASE, REM_ROWS)])


@functools.cache
def _sc_kernels():
    mesh = plsc.VectorSubcoreMesh(
        core_axis_name="c", subcore_axis_name="s",
        num_cores=NCORE, num_subcores=NSUB)
    deg = pl.kernel(
        _deg_body,
        out_type=jax.ShapeDtypeStruct((NCORE * N, HH), jnp.float32),
        mesh=mesh,
        scratch_types=[
            pltpu.VMEM((DEG_CH,), jnp.int32),
            pltpu.VMEM((DEG_CH, HH), jnp.float32),
            pltpu.VMEM((ZROWS, HH), jnp.float32),
            pltpu.VMEM_SHARED((N, HH), jnp.float32),
        ],
    )
    agg = pl.kernel(
        _agg_body,
        out_type=jax.ShapeDtypeStruct((NCORE * N, HH), jnp.float32),
        mesh=mesh,
        scratch_types=(
            [pltpu.VMEM((AGG_CH,), jnp.int32)] * (2 * NSLOT)
            + [pltpu.VMEM((AGG_CH, HH), jnp.float32)] * NSLOT
            + [pltpu.VMEM((ZROWS, HH), jnp.float32),
               pltpu.VMEM_SHARED((N, HH), jnp.float32)]
            + [pltpu.SemaphoreType.DMA] * NSLOT
        ),
    )
    return deg, agg


BLK = 1000
GRID = N // BLK


def _dinv(p_ref, q_ref):
    d = 1.0 + p_ref[:, 0:1] + q_ref[:, 0:1]
    return lax.rsqrt(d)


def _pre_body(x_ref, win_ref, bin_ref, w0_ref, p_ref, q_ref,
              h_ref, y0_ref, y1_ref):
    h = jnp.dot(x_ref[...], win_ref[...], preferred_element_type=jnp.float32,
                precision=MXU_PREC) + bin_ref[...]
    h_ref[...] = h
    dinv = _dinv(p_ref, q_ref)
    y = dinv * jnp.dot(h, w0_ref[...], preferred_element_type=jnp.float32,
                       precision=MXU_PREC)
    y0_ref[...] = y[:, :HH]
    y1_ref[...] = y[:, HH:]


def _update(h_ref, y0_ref, y1_ref, a0_ref, a1_ref, dinv, b_ref, g_ref, be_ref):
    aggy = jnp.concatenate([a0_ref[...] + y0_ref[...],
                            a1_ref[...] + y1_ref[...]], axis=1)
    u = h_ref[...] + dinv * aggy + b_ref[...]
    m = jnp.mean(u, axis=1, keepdims=True)
    v = jnp.mean((u - m) ** 2, axis=1, keepdims=True)
    hn = (u - m) * lax.rsqrt(v + EPS) * g_ref[...] + be_ref[...]
    return jnp.maximum(hn, 0.0)


def _mid_body(h_ref, y0_ref, y1_ref, a0_ref, a1_ref, p_ref, q_ref,
              b_ref, g_ref, be_ref, wn_ref, ho_ref, yo0_ref, yo1_ref):
    dinv = _dinv(p_ref, q_ref)
    h = _update(h_ref, y0_ref, y1_ref, a0_ref, a1_ref, dinv, b_ref, g_ref, be_ref)
    ho_ref[...] = h
    y = dinv * jnp.dot(h, wn_ref[...], preferred_element_type=jnp.float32,
                       precision=MXU_PREC)
    yo0_ref[...] = y[:, :HH]
    yo1_ref[...] = y[:, HH:]


def _post_body(h_ref, y0_ref, y1_ref, a0_ref, a1_ref, p_ref, q_ref,
               b_ref, g_ref, be_ref, wh1_ref, bh1_ref, wh2_ref, bh2_ref, o_ref):
    dinv = _dinv(p_ref, q_ref)
    h = _update(h_ref, y0_ref, y1_ref, a0_ref, a1_ref, dinv, b_ref, g_ref, be_ref)
    t = jnp.maximum(jnp.dot(h, wh1_ref[...], preferred_element_type=jnp.float32,
                            precision=MXU_PREC) + bh1_ref[...], 0.0)
    o = jnp.dot(t, wh2_ref[...], preferred_element_type=jnp.float32,
                precision=MXU_PREC) + bh2_ref[...]
    o_ref[...] = 1.0 / (1.0 + jnp.exp(-o))


def _row_spec(w):
    return pl.BlockSpec((BLK, w), lambda i: (i, 0))


def _hi_spec():
    return pl.BlockSpec((BLK, HH), lambda i: (i + GRID, 0))


def _full_spec(shape):
    nd = len(shape)
    return pl.BlockSpec(shape, lambda i, _nd=nd: (0,) * _nd)


_pre_call = pl.pallas_call(
    _pre_body,
    grid=(GRID,),
    in_specs=[_row_spec(IN_DIM), _full_spec((IN_DIM, H)), _full_spec((1, H)),
              _full_spec((H, H)), _row_spec(HH), _hi_spec()],
    out_specs=[_row_spec(H), _row_spec(HH), _row_spec(HH)],
    out_shape=[jax.ShapeDtypeStruct((N, H), jnp.float32),
               jax.ShapeDtypeStruct((N, HH), jnp.float32),
               jax.ShapeDtypeStruct((N, HH), jnp.float32)],
)

_mid_call = pl.pallas_call(
    _mid_body,
    grid=(GRID,),
    in_specs=[_row_spec(H), _row_spec(HH), _row_spec(HH), _row_spec(HH),
              _hi_spec(), _row_spec(HH), _hi_spec(),
              _full_spec((1, H)), _full_spec((1, H)), _full_spec((1, H)),
              _full_spec((H, H))],
    out_specs=[_row_spec(H), _row_spec(HH), _row_spec(HH)],
    out_shape=[jax.ShapeDtypeStruct((N, H), jnp.float32),
               jax.ShapeDtypeStruct((N, HH), jnp.float32),
               jax.ShapeDtypeStruct((N, HH), jnp.float32)],
)

_post_call = pl.pallas_call(
    _post_body,
    grid=(GRID,),
    in_specs=[_row_spec(H), _row_spec(HH), _row_spec(HH), _row_spec(HH),
              _hi_spec(), _row_spec(HH), _hi_spec(),
              _full_spec((1, H)), _full_spec((1, H)), _full_spec((1, H)),
              _full_spec((H, HH)), _full_spec((1, HH)),
              _full_spec((HH, 1)), _full_spec((1, 1))],
    out_specs=[_row_spec(1)],
    out_shape=[jax.ShapeDtypeStruct((N, 1), jnp.float32)],
)


def kernel(x, edge_index, W_in, b_in, W0, b0, g0, beta0, W1, b1, g1, beta1,
           W2, b2, g2, beta2, Wh1, bh1, Wh2, bh2):
    src = edge_index[0]
    dst = edge_index[1]
    _deg_kernel, _agg_kernel = _sc_kernels()

    p = _deg_kernel(dst)

    r1 = lambda a: a.reshape(1, -1)
    h0, y00, y01 = _pre_call(x, W_in, r1(b_in), W0, p, p)
    a0 = _agg_kernel(y00, y01, src, dst)
    h1, y10, y11 = _mid_call(h0, y00, y01, a0, a0, p, p,
                             r1(b0), r1(g0), r1(beta0), W1)
    a1 = _agg_kernel(y10, y11, src, dst)
    h2, y20, y21 = _mid_call(h1, y10, y11, a1, a1, p, p,
                             r1(b1), r1(g1), r1(beta1), W2)
    a2 = _agg_kernel(y20, y21, src, dst)
    (out,) = _post_call(h2, y20, y21, a2, a2, p, p,
                        r1(b2), r1(g2), r1(beta2),
                        Wh1, r1(bh1), Wh2, bh2.reshape(1, 1))
    return out[:, 0]


# speedup vs baseline: 2.1454x; 1.3915x over previous
import functools

import jax
import jax.numpy as jnp
from jax import lax
from jax.experimental import pallas as pl
from jax.experimental.pallas import tpu as pltpu
from jax.experimental.pallas import tpu_sc as plsc

N = 10000
E = 160000
IN_DIM = 128
H = 256
HH = H // 2
EPS = 1e-5

NCORE = 2
NSUB = 16
RPT = 624
REM_BASE = RPT * NSUB
REM_ROWS = N - REM_BASE
ZROWS = 16
ZCOPIES = RPT // ZROWS

E_PAD = 163840
N_ACC = N + 16

AGG_CH = 80
AGG_EPT = E // NSUB
AGG_CHUNKS = AGG_EPT // AGG_CH
NSLOT = 3
AGG_STEPS = 40

DEG_CH = 40
DEG_EPT = E // (NSUB * NCORE)
DEG_CHUNKS = DEG_EPT // DEG_CH

MXU_PREC = lax.Precision.HIGHEST


def _fill(buf, rows, width, vec):
    for j in range(rows):
        for k in range(width // 16):
            buf[j, pl.ds(k * 16, 16)] = vec


def _zero_acc(zbuf, acc, s):
    for k in range(ZCOPIES):
        pltpu.sync_copy(zbuf, acc.at[pl.ds(s * RPT + k * ZROWS, ZROWS)])

    @pl.when(s == NSUB - 1)
    def _():
        pltpu.sync_copy(zbuf.at[pl.ds(0, REM_ROWS)],
                        acc.at[pl.ds(REM_BASE, REM_ROWS)])


def _deg_body(dst_hbm, out_hbm, didx, ones, zbuf, dacc):
    c = lax.axis_index("c")
    s = lax.axis_index("s")
    _fill(ones, DEG_CH, HH, jnp.ones((16,), jnp.float32))
    _fill(zbuf, ZROWS, HH, jnp.zeros((16,), jnp.float32))
    _zero_acc(zbuf, dacc, s)
    plsc.subcore_barrier()
    ebase = (c * NSUB + s) * DEG_EPT

    def step(i, carry):
        b = ebase + i * DEG_CH
        pltpu.sync_copy(dst_hbm.at[pl.ds(b, DEG_CH)], didx)
        pltpu.sync_copy(ones, dacc.at[didx], add=True)
        return carry

    lax.fori_loop(0, DEG_CHUNKS, step, 0)
    plsc.subcore_barrier()
    pltpu.sync_copy(dacc.at[pl.ds(s * RPT, RPT)],
                    out_hbm.at[pl.ds(c * N + s * RPT, RPT)])

    @pl.when(s == NSUB - 1)
    def _():
        pltpu.sync_copy(dacc.at[pl.ds(REM_BASE, REM_ROWS)],
                        out_hbm.at[pl.ds(c * N + REM_BASE, REM_ROWS)])


def _agg_body(y0_hbm, y1_hbm, src_hbm, dst_hbm, out_hbm,
              si0, si1, si2, di0, di1, di2,
              r0, r1, r2, zbuf, acc, sg0, sg1, sg2):
    c = lax.axis_index("c")
    s = lax.axis_index("s")
    sis = [si0, si1, si2]
    dis = [di0, di1, di2]
    rows = [r0, r1, r2]
    sgs = [sg0, sg1, sg2]
    ebase = s * AGG_EPT

    def _load(g, b):
        base = ebase + g * AGG_CH
        pltpu.sync_copy(src_hbm.at[pl.ds(base, AGG_CH)], sis[b])
        pltpu.sync_copy(dst_hbm.at[pl.ds(base, AGG_CH)], dis[b])

    def _g_start(b):
        @pl.when(c == 0)
        def _():
            pltpu.async_copy(y0_hbm.at[sis[b]], rows[b], sgs[b])

        @pl.when(c == 1)
        def _():
            pltpu.async_copy(y1_hbm.at[sis[b]], rows[b], sgs[b])

    def _g_wait(b):
        @pl.when(c == 0)
        def _():
            pltpu.make_async_copy(y0_hbm.at[sis[b]], rows[b], sgs[b]).wait()

        @pl.when(c == 1)
        def _():
            pltpu.make_async_copy(y1_hbm.at[sis[b]], rows[b], sgs[b]).wait()

    def _scatter(b):
        pltpu.sync_copy(rows[b], acc.at[dis[b]], add=True)

    for b in range(NSLOT):
        _load(b, b)
        _g_start(b)
    _fill(zbuf, ZROWS, HH, jnp.zeros((16,), jnp.float32))
    _zero_acc(zbuf, acc, s)
    plsc.subcore_barrier()

    def step(t, carry):
        for b in range(NSLOT):
            _g_wait(b)
            _scatter(b)
            _load(t * NSLOT + b + NSLOT, b)
            _g_start(b)
        return carry

    lax.fori_loop(0, AGG_STEPS, step, 0)
    for b in range(NSLOT):
        _g_wait(b)
        _scatter(b)
    for g in (AGG_STEPS * NSLOT + NSLOT, AGG_STEPS * NSLOT + NSLOT + 1):
        _load(g, 0)
        _g_start(0)
        _g_wait(0)
        _scatter(0)
    plsc.subcore_barrier()
    pltpu.sync_copy(acc.at[pl.ds(s * RPT, RPT)],
                    out_hbm.at[pl.ds(c * N + s * RPT, RPT)])

    @pl.when(s == NSUB - 1)
    def _():
        pltpu.sync_copy(acc.at[pl.ds(REM_BASE, REM_ROWS)],
                        out_hbm.at[pl.ds(c * N + REM_BASE, REM_ROWS)])


@functools.cache
def _sc_kernels():
    mesh = plsc.VectorSubcoreMesh(
        core_axis_name="c", subcore_axis_name="s",
        num_cores=NCORE, num_subcores=NSUB)
    deg = pl.kernel(
        _deg_body,
        out_type=jax.ShapeDtypeStruct((NCORE * N, HH), jnp.float32),
        mesh=mesh,
        scratch_types=[
            pltpu.VMEM((DEG_CH,), jnp.int32),
            pltpu.VMEM((DEG_CH, HH), jnp.float32),
            pltpu.VMEM((ZROWS, HH), jnp.float32),
            pltpu.VMEM_SHARED((N, HH), jnp.float32),
        ],
    )
    agg = pl.kernel(
        _agg_body,
        out_type=jax.ShapeDtypeStruct((NCORE * N, HH), jnp.float32),
        mesh=mesh,
        scratch_types=(
            [pltpu.VMEM((AGG_CH,), jnp.int32)] * (2 * NSLOT)
            + [pltpu.VMEM((AGG_CH, HH), jnp.float32)] * NSLOT
            + [pltpu.VMEM((ZROWS, HH), jnp.float32),
               pltpu.VMEM_SHARED((N, HH), jnp.float32)]
            + [pltpu.SemaphoreType.DMA] * NSLOT
        ),
    )
    return deg, agg


BLK = 1000
GRID = N // BLK


def _dinv(p_ref, q_ref):
    d = 1.0 + p_ref[:, 0:1] + q_ref[:, 0:1]
    return lax.rsqrt(d)


def _pre_body(x_ref, win_ref, bin_ref, w0_ref, p_ref, q_ref,
              h_ref, y0_ref, y1_ref):
    h = jnp.dot(x_ref[...], win_ref[...], preferred_element_type=jnp.float32,
                precision=MXU_PREC) + bin_ref[...]
    h_ref[...] = h
    dinv = _dinv(p_ref, q_ref)
    y = dinv * jnp.dot(h, w0_ref[...], preferred_element_type=jnp.float32,
                       precision=MXU_PREC)
    y0_ref[...] = y[:, :HH]
    y1_ref[...] = y[:, HH:]


def _update(h_ref, y0_ref, y1_ref, a0_ref, a1_ref, dinv, b_ref, g_ref, be_ref):
    aggy = jnp.concatenate([a0_ref[...] + y0_ref[...],
                            a1_ref[...] + y1_ref[...]], axis=1)
    u = h_ref[...] + dinv * aggy + b_ref[...]
    m = jnp.mean(u, axis=1, keepdims=True)
    v = jnp.mean((u - m) ** 2, axis=1, keepdims=True)
    hn = (u - m) * lax.rsqrt(v + EPS) * g_ref[...] + be_ref[...]
    return jnp.maximum(hn, 0.0)


def _mid_body(h_ref, y0_ref, y1_ref, a0_ref, a1_ref, p_ref, q_ref,
              b_ref, g_ref, be_ref, wn_ref, ho_ref, yo0_ref, yo1_ref):
    dinv = _dinv(p_ref, q_ref)
    h = _update(h_ref, y0_ref, y1_ref, a0_ref, a1_ref, dinv, b_ref, g_ref, be_ref)
    ho_ref[...] = h
    y = dinv * jnp.dot(h, wn_ref[...], preferred_element_type=jnp.float32,
                       precision=MXU_PREC)
    yo0_ref[...] = y[:, :HH]
    yo1_ref[...] = y[:, HH:]


def _post_body(h_ref, y0_ref, y1_ref, a0_ref, a1_ref, p_ref, q_ref,
               b_ref, g_ref, be_ref, wh1_ref, bh1_ref, wh2_ref, bh2_ref, o_ref):
    dinv = _dinv(p_ref, q_ref)
    h = _update(h_ref, y0_ref, y1_ref, a0_ref, a1_ref, dinv, b_ref, g_ref, be_ref)
    t = jnp.maximum(jnp.dot(h, wh1_ref[...], preferred_element_type=jnp.float32,
                            precision=MXU_PREC) + bh1_ref[...], 0.0)
    o = jnp.dot(t, wh2_ref[...], preferred_element_type=jnp.float32,
                precision=MXU_PREC) + bh2_ref[...]
    o_ref[...] = 1.0 / (1.0 + jnp.exp(-o))


def _row_spec(w):
    return pl.BlockSpec((BLK, w), lambda i: (i, 0))


def _hi_spec():
    return pl.BlockSpec((BLK, HH), lambda i: (i + GRID, 0))


def _full_spec(shape):
    nd = len(shape)
    return pl.BlockSpec(shape, lambda i, _nd=nd: (0,) * _nd)


_pre_call = pl.pallas_call(
    _pre_body,
    grid=(GRID,),
    in_specs=[_row_spec(IN_DIM), _full_spec((IN_DIM, H)), _full_spec((1, H)),
              _full_spec((H, H)), _row_spec(HH), _hi_spec()],
    out_specs=[_row_spec(H), _row_spec(HH), _row_spec(HH)],
    out_shape=[jax.ShapeDtypeStruct((N, H), jnp.float32),
               jax.ShapeDtypeStruct((N, HH), jnp.float32),
               jax.ShapeDtypeStruct((N, HH), jnp.float32)],
)

_mid_call = pl.pallas_call(
    _mid_body,
    grid=(GRID,),
    in_specs=[_row_spec(H), _row_spec(HH), _row_spec(HH), _row_spec(HH),
              _hi_spec(), _row_spec(HH), _hi_spec(),
              _full_spec((1, H)), _full_spec((1, H)), _full_spec((1, H)),
              _full_spec((H, H))],
    out_specs=[_row_spec(H), _row_spec(HH), _row_spec(HH)],
    out_shape=[jax.ShapeDtypeStruct((N, H), jnp.float32),
               jax.ShapeDtypeStruct((N, HH), jnp.float32),
               jax.ShapeDtypeStruct((N, HH), jnp.float32)],
)

_post_call = pl.pallas_call(
    _post_body,
    grid=(GRID,),
    in_specs=[_row_spec(H), _row_spec(HH), _row_spec(HH), _row_spec(HH),
              _hi_spec(), _row_spec(HH), _hi_spec(),
              _full_spec((1, H)), _full_spec((1, H)), _full_spec((1, H)),
              _full_spec((H, HH)), _full_spec((1, HH)),
              _full_spec((HH, 1)), _full_spec((1, 1))],
    out_specs=[_row_spec(1)],
    out_shape=[jax.ShapeDtypeStruct((N, 1), jnp.float32)],
)


def kernel(x, edge_index, W_in, b_in, W0, b0, g0, beta0, W1, b1, g1, beta1,
           W2, b2, g2, beta2, Wh1, bh1, Wh2, bh2):
    src = edge_index[0]
    dst = edge_index[1]
    _deg_kernel, _agg_kernel = _sc_kernels()

    p = _deg_kernel(dst)

    r1 = lambda a: a.reshape(1, -1)
    h0, y00, y01 = _pre_call(x, W_in, r1(b_in), W0, p, p)
    a0 = _agg_kernel(y00, y01, src, dst)
    h1, y10, y11 = _mid_call(h0, y00, y01, a0, a0, p, p,
                             r1(b0), r1(g0), r1(beta0), W1)
    a1 = _agg_kernel(y10, y11, src, dst)
    h2, y20, y21 = _mid_call(h1, y10, y11, a1, a1, p, p,
                             r1(b1), r1(g1), r1(beta1), W2)
    a2 = _agg_kernel(y20, y21, src, dst)
    (out,) = _post_call(h2, y20, y21, a2, a2, p, p,
                        r1(b2), r1(g2), r1(beta2),
                        Wh1, r1(bh1), Wh2, bh2.reshape(1, 1))
    return out[:, 0]


# speedup vs baseline: 2.2262x; 1.0377x over previous
import functools

import jax
import jax.numpy as jnp
from jax import lax
from jax.experimental import pallas as pl
from jax.experimental.pallas import tpu as pltpu
from jax.experimental.pallas import tpu_sc as plsc

N = 10000
E = 160000
IN_DIM = 128
H = 256
HH = H // 2
EPS = 1e-5

NCORE = 2
NSUB = 16
RPT = 624
REM_BASE = RPT * NSUB
REM_ROWS = N - REM_BASE
ZROWS = 16
ZCOPIES = RPT // ZROWS

E_PAD = 163840
N_ACC = N + 16

AGG_CH = 80
AGG_EPT = E // NSUB
AGG_CHUNKS = AGG_EPT // AGG_CH
NSLOT = 3
AGG_STEPS = 40

DEG_CH = 40
DEG_EPT = E // (NSUB * NCORE)
DEG_CHUNKS = DEG_EPT // DEG_CH

MXU_PREC = lax.Precision.HIGHEST


def _fill(buf, rows, width, vec):
    for j in range(rows):
        for k in range(width // 16):
            buf[j, pl.ds(k * 16, 16)] = vec


def _zero_acc(zbuf, acc, s):
    for k in range(ZCOPIES):
        pltpu.sync_copy(zbuf, acc.at[pl.ds(s * RPT + k * ZROWS, ZROWS)])

    @pl.when(s == NSUB - 1)
    def _():
        pltpu.sync_copy(zbuf.at[pl.ds(0, REM_ROWS)],
                        acc.at[pl.ds(REM_BASE, REM_ROWS)])


def _deg_body(dst_hbm, out_hbm, didx, didx1, didx2, ones, zbuf, dacc,
              ss0, ss1, ss2):
    c = lax.axis_index("c")
    s = lax.axis_index("s")
    dis = [didx, didx1, didx2]
    sss = [ss0, ss1, ss2]
    _fill(ones, DEG_CH, HH, jnp.ones((16,), jnp.float32))
    _fill(zbuf, ZROWS, HH, jnp.zeros((16,), jnp.float32))
    ebase = (c * NSUB + s) * DEG_EPT

    def _load(g, b):
        pltpu.sync_copy(dst_hbm.at[pl.ds(ebase + g * DEG_CH, DEG_CH)], dis[b])

    def _s_start(b):
        pltpu.async_copy(ones, dacc.at[dis[b]], sss[b], add=True)

    def _s_wait(b):
        pltpu.make_async_copy(ones, dacc.at[dis[b]], sss[b]).wait()

    for b in range(3):
        _load(b, b)
    _zero_acc(zbuf, dacc, s)
    plsc.subcore_barrier()
    for b in range(3):
        _s_start(b)

    def step(t, carry):
        for b in range(3):
            g = t * 3 + b
            _s_wait(b)
            _load(g + 3, b)
            _s_start(b)
        return carry

    lax.fori_loop(0, 40, step, 0)
    for b in range(3):
        _s_wait(b)
    for g in (123, 124):
        _load(g, 0)
        _s_start(0)
        _s_wait(0)
    plsc.subcore_barrier()
    pltpu.sync_copy(dacc.at[pl.ds(s * RPT, RPT)],
                    out_hbm.at[pl.ds(c * N + s * RPT, RPT)])

    @pl.when(s == NSUB - 1)
    def _():
        pltpu.sync_copy(dacc.at[pl.ds(REM_BASE, REM_ROWS)],
                        out_hbm.at[pl.ds(c * N + REM_BASE, REM_ROWS)])


def _agg_body(y0_hbm, y1_hbm, src_hbm, dst_hbm, out_hbm,
              si0, si1, si2, di0, di1, di2,
              r0, r1, r2, zbuf, acc, sg0, sg1, sg2):
    c = lax.axis_index("c")
    s = lax.axis_index("s")
    sis = [si0, si1, si2]
    dis = [di0, di1, di2]
    rows = [r0, r1, r2]
    sgs = [sg0, sg1, sg2]
    ebase = s * AGG_EPT

    def _load(g, b):
        base = ebase + g * AGG_CH
        pltpu.sync_copy(src_hbm.at[pl.ds(base, AGG_CH)], sis[b])
        pltpu.sync_copy(dst_hbm.at[pl.ds(base, AGG_CH)], dis[b])

    def _g_start(b):
        @pl.when(c == 0)
        def _():
            pltpu.async_copy(y0_hbm.at[sis[b]], rows[b], sgs[b])

        @pl.when(c == 1)
        def _():
            pltpu.async_copy(y1_hbm.at[sis[b]], rows[b], sgs[b])

    def _g_wait(b):
        @pl.when(c == 0)
        def _():
            pltpu.make_async_copy(y0_hbm.at[sis[b]], rows[b], sgs[b]).wait()

        @pl.when(c == 1)
        def _():
            pltpu.make_async_copy(y1_hbm.at[sis[b]], rows[b], sgs[b]).wait()

    def _scatter(b):
        pltpu.sync_copy(rows[b], acc.at[dis[b]], add=True)

    for b in range(NSLOT):
        _load(b, b)
        _g_start(b)
    _fill(zbuf, ZROWS, HH, jnp.zeros((16,), jnp.float32))
    _zero_acc(zbuf, acc, s)
    plsc.subcore_barrier()

    def step(t, carry):
        for b in range(NSLOT):
            _g_wait(b)
            _scatter(b)
            _load(t * NSLOT + b + NSLOT, b)
            _g_start(b)
        return carry

    lax.fori_loop(0, AGG_STEPS, step, 0)
    for b in range(NSLOT):
        _g_wait(b)
        _scatter(b)
    for g in (AGG_STEPS * NSLOT + NSLOT, AGG_STEPS * NSLOT + NSLOT + 1):
        _load(g, 0)
        _g_start(0)
        _g_wait(0)
        _scatter(0)
    plsc.subcore_barrier()
    pltpu.sync_copy(acc.at[pl.ds(s * RPT, RPT)],
                    out_hbm.at[pl.ds(c * N + s * RPT, RPT)])

    @pl.when(s == NSUB - 1)
    def _():
        pltpu.sync_copy(acc.at[pl.ds(REM_BASE, REM_ROWS)],
                        out_hbm.at[pl.ds(c * N + REM_BASE, REM_ROWS)])


@functools.cache
def _sc_kernels():
    mesh = plsc.VectorSubcoreMesh(
        core_axis_name="c", subcore_axis_name="s",
        num_cores=NCORE, num_subcores=NSUB)
    deg = pl.kernel(
        _deg_body,
        out_type=jax.ShapeDtypeStruct((NCORE * N, HH), jnp.float32),
        mesh=mesh,
        scratch_types=[
            pltpu.VMEM((DEG_CH,), jnp.int32),
            pltpu.VMEM((DEG_CH,), jnp.int32),
            pltpu.VMEM((DEG_CH,), jnp.int32),
            pltpu.VMEM((DEG_CH, HH), jnp.float32),
            pltpu.VMEM((ZROWS, HH), jnp.float32),
            pltpu.VMEM_SHARED((N, HH), jnp.float32),
            pltpu.SemaphoreType.DMA,
            pltpu.SemaphoreType.DMA,
            pltpu.SemaphoreType.DMA,
        ],
    )
    agg = pl.kernel(
        _agg_body,
        out_type=jax.ShapeDtypeStruct((NCORE * N, HH), jnp.float32),
        mesh=mesh,
        scratch_types=(
            [pltpu.VMEM((AGG_CH,), jnp.int32)] * (2 * NSLOT)
            + [pltpu.VMEM((AGG_CH, HH), jnp.float32)] * NSLOT
            + [pltpu.VMEM((ZROWS, HH), jnp.float32),
               pltpu.VMEM_SHARED((N, HH), jnp.float32)]
            + [pltpu.SemaphoreType.DMA] * NSLOT
        ),
    )
    return deg, agg


BLK = 1000
GRID = N // BLK


def _dinv(p_ref, q_ref):
    d = 1.0 + p_ref[:, 0:1] + q_ref[:, 0:1]
    return lax.rsqrt(d)


def _pre_body(x_ref, win_ref, bin_ref, w0_ref, p_ref, q_ref,
              h_ref, y0_ref, y1_ref):
    h = jnp.dot(x_ref[...], win_ref[...], preferred_element_type=jnp.float32,
                precision=MXU_PREC) + bin_ref[...]
    h_ref[...] = h
    dinv = _dinv(p_ref, q_ref)
    y = dinv * jnp.dot(h, w0_ref[...], preferred_element_type=jnp.float32,
                       precision=MXU_PREC)
    y0_ref[...] = y[:, :HH]
    y1_ref[...] = y[:, HH:]


def _update(h_ref, y0_ref, y1_ref, a0_ref, a1_ref, dinv, b_ref, g_ref, be_ref):
    aggy = jnp.concatenate([a0_ref[...] + y0_ref[...],
                            a1_ref[...] + y1_ref[...]], axis=1)
    u = h_ref[...] + dinv * aggy + b_ref[...]
    m = jnp.mean(u, axis=1, keepdims=True)
    v = jnp.mean((u - m) ** 2, axis=1, keepdims=True)
    hn = (u - m) * lax.rsqrt(v + EPS) * g_ref[...] + be_ref[...]
    return jnp.maximum(hn, 0.0)


def _mid_body(h_ref, y0_ref, y1_ref, a0_ref, a1_ref, p_ref, q_ref,
              b_ref, g_ref, be_ref, wn_ref, ho_ref, yo0_ref, yo1_ref):
    dinv = _dinv(p_ref, q_ref)
    h = _update(h_ref, y0_ref, y1_ref, a0_ref, a1_ref, dinv, b_ref, g_ref, be_ref)
    ho_ref[...] = h
    y = dinv * jnp.dot(h, wn_ref[...], preferred_element_type=jnp.float32,
                       precision=MXU_PREC)
    yo0_ref[...] = y[:, :HH]
    yo1_ref[...] = y[:, HH:]


def _post_body(h_ref, y0_ref, y1_ref, a0_ref, a1_ref, p_ref, q_ref,
               b_ref, g_ref, be_ref, wh1_ref, bh1_ref, wh2_ref, bh2_ref, o_ref):
    dinv = _dinv(p_ref, q_ref)
    h = _update(h_ref, y0_ref, y1_ref, a0_ref, a1_ref, dinv, b_ref, g_ref, be_ref)
    t = jnp.maximum(jnp.dot(h, wh1_ref[...], preferred_element_type=jnp.float32,
                            precision=MXU_PREC) + bh1_ref[...], 0.0)
    o = jnp.dot(t, wh2_ref[...], preferred_element_type=jnp.float32,
                precision=MXU_PREC) + bh2_ref[...]
    o_ref[...] = 1.0 / (1.0 + jnp.exp(-o))


def _row_spec(w):
    return pl.BlockSpec((BLK, w), lambda i: (i, 0))


def _hi_spec():
    return pl.BlockSpec((BLK, HH), lambda i: (i + GRID, 0))


def _full_spec(shape):
    nd = len(shape)
    return pl.BlockSpec(shape, lambda i, _nd=nd: (0,) * _nd)


_pre_call = pl.pallas_call(
    _pre_body,
    grid=(GRID,),
    in_specs=[_row_spec(IN_DIM), _full_spec((IN_DIM, H)), _full_spec((1, H)),
              _full_spec((H, H)), _row_spec(HH), _hi_spec()],
    out_specs=[_row_spec(H), _row_spec(HH), _row_spec(HH)],
    out_shape=[jax.ShapeDtypeStruct((N, H), jnp.float32),
               jax.ShapeDtypeStruct((N, HH), jnp.float32),
               jax.ShapeDtypeStruct((N, HH), jnp.float32)],
)

_mid_call = pl.pallas_call(
    _mid_body,
    grid=(GRID,),
    in_specs=[_row_spec(H), _row_spec(HH), _row_spec(HH), _row_spec(HH),
              _hi_spec(), _row_spec(HH), _hi_spec(),
              _full_spec((1, H)), _full_spec((1, H)), _full_spec((1, H)),
              _full_spec((H, H))],
    out_specs=[_row_spec(H), _row_spec(HH), _row_spec(HH)],
    out_shape=[jax.ShapeDtypeStruct((N, H), jnp.float32),
               jax.ShapeDtypeStruct((N, HH), jnp.float32),
               jax.ShapeDtypeStruct((N, HH), jnp.float32)],
)

_post_call = pl.pallas_call(
    _post_body,
    grid=(GRID,),
    in_specs=[_row_spec(H), _row_spec(HH), _row_spec(HH), _row_spec(HH),
              _hi_spec(), _row_spec(HH), _hi_spec(),
              _full_spec((1, H)), _full_spec((1, H)), _full_spec((1, H)),
              _full_spec((H, HH)), _full_spec((1, HH)),
              _full_spec((HH, 1)), _full_spec((1, 1))],
    out_specs=[_row_spec(1)],
    out_shape=[jax.ShapeDtypeStruct((N, 1), jnp.float32)],
)


def kernel(x, edge_index, W_in, b_in, W0, b0, g0, beta0, W1, b1, g1, beta1,
           W2, b2, g2, beta2, Wh1, bh1, Wh2, bh2):
    src = edge_index[0]
    dst = edge_index[1]
    _deg_kernel, _agg_kernel = _sc_kernels()

    p = _deg_kernel(dst)

    r1 = lambda a: a.reshape(1, -1)
    h0, y00, y01 = _pre_call(x, W_in, r1(b_in), W0, p, p)
    a0 = _agg_kernel(y00, y01, src, dst)
    h1, y10, y11 = _mid_call(h0, y00, y01, a0, a0, p, p,
                             r1(b0), r1(g0), r1(beta0), W1)
    a1 = _agg_kernel(y10, y11, src, dst)
    h2, y20, y21 = _mid_call(h1, y10, y11, a1, a1, p, p,
                             r1(b1), r1(g1), r1(beta1), W2)
    a2 = _agg_kernel(y20, y21, src, dst)
    (out,) = _post_call(h2, y20, y21, a2, a2, p, p,
                        r1(b2), r1(g2), r1(beta2),
                        Wh1, r1(bh1), Wh2, bh2.reshape(1, 1))
    return out[:, 0]


# speedup vs baseline: 2.2268x; 1.0003x over previous
import functools

import jax
import jax.numpy as jnp
from jax import lax
from jax.experimental import pallas as pl
from jax.experimental.pallas import tpu as pltpu
from jax.experimental.pallas import tpu_sc as plsc

N = 10000
E = 160000
IN_DIM = 128
H = 256
HH = H // 2
EPS = 1e-5

NCORE = 2
NSUB = 16
RPT = 624
REM_BASE = RPT * NSUB
REM_ROWS = N - REM_BASE
ZROWS = 16
ZCOPIES = RPT // ZROWS

AGG_CH = 80
AGG_EPT = E // NSUB
AGG_CHUNKS = AGG_EPT // AGG_CH
NSLOT = 3
AGG_STEPS = 40

DEG_CH = 40
DEG_EPT = E // (NSUB * NCORE)
DEG_CHUNKS = DEG_EPT // DEG_CH

MXU_PREC = lax.Precision.HIGHEST


def _fill(buf, rows, width, vec):
    for j in range(rows):
        for k in range(width // 16):
            buf[j, pl.ds(k * 16, 16)] = vec


def _zero_acc(zbuf, acc, s):
    for k in range(ZCOPIES):
        pltpu.sync_copy(zbuf, acc.at[pl.ds(s * RPT + k * ZROWS, ZROWS)])

    @pl.when(s == NSUB - 1)
    def _():
        pltpu.sync_copy(zbuf.at[pl.ds(0, REM_ROWS)],
                        acc.at[pl.ds(REM_BASE, REM_ROWS)])


def _deg_body(dst_hbm, out_hbm, didx, didx1, didx2, ones, zbuf, dacc,
              ss0, ss1, ss2):
    c = lax.axis_index("c")
    s = lax.axis_index("s")
    dis = [didx, didx1, didx2]
    sss = [ss0, ss1, ss2]
    _fill(ones, DEG_CH, HH, jnp.ones((16,), jnp.float32))
    _fill(zbuf, ZROWS, HH, jnp.zeros((16,), jnp.float32))
    ebase = (c * NSUB + s) * DEG_EPT

    def _load(g, b):
        pltpu.sync_copy(dst_hbm.at[pl.ds(ebase + g * DEG_CH, DEG_CH)], dis[b])

    def _s_start(b):
        pltpu.async_copy(ones, dacc.at[dis[b]], sss[b], add=True)

    def _s_wait(b):
        pltpu.make_async_copy(ones, dacc.at[dis[b]], sss[b]).wait()

    for b in range(3):
        _load(b, b)
    _zero_acc(zbuf, dacc, s)
    plsc.subcore_barrier()
    for b in range(3):
        _s_start(b)

    def step(t, carry):
        for b in range(3):
            g = t * 3 + b
            _s_wait(b)
            _load(g + 3, b)
            _s_start(b)
        return carry

    lax.fori_loop(0, 40, step, 0)
    for b in range(3):
        _s_wait(b)
    for g in (123, 124):
        _load(g, 0)
        _s_start(0)
        _s_wait(0)
    plsc.subcore_barrier()
    pltpu.sync_copy(dacc.at[pl.ds(s * RPT, RPT)],
                    out_hbm.at[pl.ds(c * N + s * RPT, RPT)])

    @pl.when(s == NSUB - 1)
    def _():
        pltpu.sync_copy(dacc.at[pl.ds(REM_BASE, REM_ROWS)],
                        out_hbm.at[pl.ds(c * N + REM_BASE, REM_ROWS)])


def _agg_body(y0_hbm, y1_hbm, src_hbm, dst_hbm, out_hbm,
              si0, si1, si2, di0, di1, di2,
              r0, r1, r2, zbuf, acc, sg0, sg1, sg2):
    c = lax.axis_index("c")
    s = lax.axis_index("s")
    sis = [si0, si1, si2]
    dis = [di0, di1, di2]
    rows = [r0, r1, r2]
    sgs = [sg0, sg1, sg2]
    ebase = s * AGG_EPT

    def _load(g, b):
        base = ebase + g * AGG_CH
        pltpu.sync_copy(src_hbm.at[pl.ds(base, AGG_CH)], sis[b])
        pltpu.sync_copy(dst_hbm.at[pl.ds(base, AGG_CH)], dis[b])

    def _g_start(b):
        @pl.when(c == 0)
        def _():
            pltpu.async_copy(y0_hbm.at[sis[b]], rows[b], sgs[b])

        @pl.when(c == 1)
        def _():
            pltpu.async_copy(y1_hbm.at[sis[b]], rows[b], sgs[b])

    def _g_wait(b):
        @pl.when(c == 0)
        def _():
            pltpu.make_async_copy(y0_hbm.at[sis[b]], rows[b], sgs[b]).wait()

        @pl.when(c == 1)
        def _():
            pltpu.make_async_copy(y1_hbm.at[sis[b]], rows[b], sgs[b]).wait()

    def _scatter(b):
        pltpu.sync_copy(rows[b], acc.at[dis[b]], add=True)

    for b in range(NSLOT):
        _load(b, b)
        _g_start(b)
    _fill(zbuf, ZROWS, HH, jnp.zeros((16,), jnp.float32))
    _zero_acc(zbuf, acc, s)
    plsc.subcore_barrier()

    def step(t, carry):
        for b in range(NSLOT):
            _g_wait(b)
            _scatter(b)
            _load(t * NSLOT + b + NSLOT, b)
            _g_start(b)
        return carry

    lax.fori_loop(0, AGG_STEPS, step, 0)
    for b in range(NSLOT):
        _g_wait(b)
        _scatter(b)
    for g in (AGG_STEPS * NSLOT + NSLOT, AGG_STEPS * NSLOT + NSLOT + 1):
        _load(g, 0)
        _g_start(0)
        _g_wait(0)
        _scatter(0)
    plsc.subcore_barrier()
    pltpu.sync_copy(acc.at[pl.ds(s * RPT, RPT)],
                    out_hbm.at[pl.ds(c * N + s * RPT, RPT)])

    @pl.when(s == NSUB - 1)
    def _():
        pltpu.sync_copy(acc.at[pl.ds(REM_BASE, REM_ROWS)],
                        out_hbm.at[pl.ds(c * N + REM_BASE, REM_ROWS)])


@functools.cache
def _sc_kernels():
    mesh = plsc.VectorSubcoreMesh(
        core_axis_name="c", subcore_axis_name="s",
        num_cores=NCORE, num_subcores=NSUB)
    deg = pl.kernel(
        _deg_body,
        out_type=jax.ShapeDtypeStruct((NCORE * N, HH), jnp.float32),
        mesh=mesh,
        scratch_types=[
            pltpu.VMEM((DEG_CH,), jnp.int32),
            pltpu.VMEM((DEG_CH,), jnp.int32),
            pltpu.VMEM((DEG_CH,), jnp.int32),
            pltpu.VMEM((DEG_CH, HH), jnp.float32),
            pltpu.VMEM((ZROWS, HH), jnp.float32),
            pltpu.VMEM_SHARED((N, HH), jnp.float32),
            pltpu.SemaphoreType.DMA,
            pltpu.SemaphoreType.DMA,
            pltpu.SemaphoreType.DMA,
        ],
    )
    agg = pl.kernel(
        _agg_body,
        out_type=jax.ShapeDtypeStruct((NCORE * N, HH), jnp.float32),
        mesh=mesh,
        scratch_types=(
            [pltpu.VMEM((AGG_CH,), jnp.int32)] * (2 * NSLOT)
            + [pltpu.VMEM((AGG_CH, HH), jnp.float32)] * NSLOT
            + [pltpu.VMEM((ZROWS, HH), jnp.float32),
               pltpu.VMEM_SHARED((N, HH), jnp.float32)]
            + [pltpu.SemaphoreType.DMA] * NSLOT
        ),
    )
    return deg, agg


BLK = 1000
GRID = N // BLK


def _dinv(p_ref, q_ref):
    d = 1.0 + p_ref[:, 0:1] + q_ref[:, 0:1]
    return lax.rsqrt(d)


def _pre_body(x_ref, win_ref, bin_ref, w0_ref, p_ref, q_ref,
              h_ref, y0_ref, y1_ref):
    h = jnp.dot(x_ref[...], win_ref[...], preferred_element_type=jnp.float32,
                precision=MXU_PREC) + bin_ref[...]
    h_ref[...] = h
    dinv = _dinv(p_ref, q_ref)
    y = dinv * jnp.dot(h, w0_ref[...], preferred_element_type=jnp.float32,
                       precision=MXU_PREC)
    y0_ref[...] = y[:, :HH]
    y1_ref[...] = y[:, HH:]


def _update(h_ref, y0_ref, y1_ref, a0_ref, a1_ref, dinv, b_ref, g_ref, be_ref):
    aggy = jnp.concatenate([a0_ref[...] + y0_ref[...],
                            a1_ref[...] + y1_ref[...]], axis=1)
    u = h_ref[...] + dinv * aggy + b_ref[...]
    m = jnp.mean(u, axis=1, keepdims=True)
    v = jnp.mean((u - m) ** 2, axis=1, keepdims=True)
    hn = (u - m) * lax.rsqrt(v + EPS) * g_ref[...] + be_ref[...]
    return jnp.maximum(hn, 0.0)


def _mid_body(h_ref, y0_ref, y1_ref, a0_ref, a1_ref, p_ref, q_ref,
              b_ref, g_ref, be_ref, wn_ref, ho_ref, yo0_ref, yo1_ref):
    dinv = _dinv(p_ref, q_ref)
    h = _update(h_ref, y0_ref, y1_ref, a0_ref, a1_ref, dinv, b_ref, g_ref, be_ref)
    ho_ref[...] = h
    y = dinv * jnp.dot(h, wn_ref[...], preferred_element_type=jnp.float32,
                       precision=MXU_PREC)
    yo0_ref[...] = y[:, :HH]
    yo1_ref[...] = y[:, HH:]


def _post_body(h_ref, y0_ref, y1_ref, a0_ref, a1_ref, p_ref, q_ref,
               b_ref, g_ref, be_ref, wh1_ref, bh1_ref, wh2_ref, bh2_ref, o_ref):
    dinv = _dinv(p_ref, q_ref)
    h = _update(h_ref, y0_ref, y1_ref, a0_ref, a1_ref, dinv, b_ref, g_ref, be_ref)
    t = jnp.maximum(jnp.dot(h, wh1_ref[...], preferred_element_type=jnp.float32,
                            precision=MXU_PREC) + bh1_ref[...], 0.0)
    o = jnp.dot(t, wh2_ref[...], preferred_element_type=jnp.float32,
                precision=MXU_PREC) + bh2_ref[...]
    o_ref[...] = 1.0 / (1.0 + jnp.exp(-o))


def _row_spec(w):
    return pl.BlockSpec((BLK, w), lambda i: (i, 0))


def _hi_spec():
    return pl.BlockSpec((BLK, HH), lambda i: (i + GRID, 0))


def _full_spec(shape):
    nd = len(shape)
    return pl.BlockSpec(shape, lambda i, _nd=nd: (0,) * _nd)


_pre_call = pl.pallas_call(
    _pre_body,
    grid=(GRID,),
    in_specs=[_row_spec(IN_DIM), _full_spec((IN_DIM, H)), _full_spec((1, H)),
              _full_spec((H, H)), _row_spec(HH), _hi_spec()],
    out_specs=[_row_spec(H), _row_spec(HH), _row_spec(HH)],
    out_shape=[jax.ShapeDtypeStruct((N, H), jnp.float32),
               jax.ShapeDtypeStruct((N, HH), jnp.float32),
               jax.ShapeDtypeStruct((N, HH), jnp.float32)],
)

_mid_call = pl.pallas_call(
    _mid_body,
    grid=(GRID,),
    in_specs=[_row_spec(H), _row_spec(HH), _row_spec(HH), _row_spec(HH),
              _hi_spec(), _row_spec(HH), _hi_spec(),
              _full_spec((1, H)), _full_spec((1, H)), _full_spec((1, H)),
              _full_spec((H, H))],
    out_specs=[_row_spec(H), _row_spec(HH), _row_spec(HH)],
    out_shape=[jax.ShapeDtypeStruct((N, H), jnp.float32),
               jax.ShapeDtypeStruct((N, HH), jnp.float32),
               jax.ShapeDtypeStruct((N, HH), jnp.float32)],
)

_post_call = pl.pallas_call(
    _post_body,
    grid=(GRID,),
    in_specs=[_row_spec(H), _row_spec(HH), _row_spec(HH), _row_spec(HH),
              _hi_spec(), _row_spec(HH), _hi_spec(),
              _full_spec((1, H)), _full_spec((1, H)), _full_spec((1, H)),
              _full_spec((H, HH)), _full_spec((1, HH)),
              _full_spec((HH, 1)), _full_spec((1, 1))],
    out_specs=[_row_spec(1)],
    out_shape=[jax.ShapeDtypeStruct((N, 1), jnp.float32)],
)


def kernel(x, edge_index, W_in, b_in, W0, b0, g0, beta0, W1, b1, g1, beta1,
           W2, b2, g2, beta2, Wh1, bh1, Wh2, bh2):
    src = edge_index[0]
    dst = edge_index[1]
    _deg_kernel, _agg_kernel = _sc_kernels()

    p = _deg_kernel(dst)

    r1 = lambda a: a.reshape(1, -1)
    h0, y00, y01 = _pre_call(x, W_in, r1(b_in), W0, p, p)
    a0 = _agg_kernel(y00, y01, src, dst)
    h1, y10, y11 = _mid_call(h0, y00, y01, a0, a0, p, p,
                             r1(b0), r1(g0), r1(beta0), W1)
    a1 = _agg_kernel(y10, y11, src, dst)
    h2, y20, y21 = _mid_call(h1, y10, y11, a1, a1, p, p,
                             r1(b1), r1(g1), r1(beta1), W2)
    a2 = _agg_kernel(y20, y21, src, dst)
    (out,) = _post_call(h2, y20, y21, a2, a2, p, p,
                        r1(b2), r1(g2), r1(beta2),
                        Wh1, r1(bh1), Wh2, bh2.reshape(1, 1))
    return out[:, 0]
